# Initial kernel scaffold; baseline (speedup 1.0000x reference)
import jax
import jax.numpy as jnp
from jax.experimental import pallas as pl


def kernel(rpn_feature, anchors, img_sz, W_cls, b_cls, W_bbox, b_bbox):
    raise NotImplementedError("write your pallas kernel here")


# speedup vs baseline: 17.2335x; 17.2335x over previous
import functools

import jax
import jax.numpy as jnp
from jax.experimental import pallas as pl
from jax.experimental.pallas import tpu as pltpu

ANCHORS_NUM = 9
NMS_OUT = 300
IOU_THR = 0.7

N_PIX = 2304
C_IN = 512
N_BOX = N_PIX * ANCHORS_NUM
ROWS = 168
LANES = 128
N_PAD = ROWS * LANES - N_BOX
NEG_INF = float("-inf")


def _head_body(flat_ref, wc_ref, bc_ref, wb_ref, bb_ref, prob_ref, bbox_ref):
    flat = flat_ref[...]
    cls = jnp.dot(flat, wc_ref[...], preferred_element_type=jnp.float32) + bc_ref[...]
    left = jnp.concatenate([cls[:, 1:], cls[:, :1]], axis=1)
    right = jnp.concatenate([cls[:, -1:], cls[:, :-1]], axis=1)
    col = jax.lax.broadcasted_iota(jnp.int32, cls.shape, 1)
    partner = jnp.where(col % 2 == 0, left, right)
    m = jnp.maximum(cls, partner)
    e = jnp.exp(cls - m)
    ep = jnp.exp(partner - m)
    prob_ref[...] = e / (e + ep)
    bbox_ref[...] = (
        jnp.dot(flat, wb_ref[...], preferred_element_type=jnp.float32) + bb_ref[...]
    )


def _nms_body(anc_ref, t_ref, s_in_ref, img_ref,
              keep_ref,
              x0_ref, y0_ref, x1_ref, y1_ref, ar_ref, s_ref):
    a0 = anc_ref[0]
    a1 = anc_ref[1]
    a2 = anc_ref[2]
    a3 = anc_ref[3]
    t0 = t_ref[0]
    t1 = t_ref[1]
    t3 = t_ref[2]
    w = a3 - a1 + 1.0
    h = a2 - a0 + 1.0
    x = a0 + 0.5 * h
    y = a1 + 0.5 * w
    x_pred = t0 * h + x
    y_pred = t1 * w + y
    h_pred = jnp.exp(t3) * h
    x0 = x_pred - 0.5 * h_pred
    x1 = x_pred + 0.5 * h_pred
    y0 = y_pred - 0.5 * y_pred
    y1 = y_pred + 0.5 * y_pred
    x0 = jnp.maximum(x0, 0.0)
    x1 = jnp.minimum(x1, img_ref[0])
    y0 = jnp.maximum(y0, 0.0)
    y1 = jnp.minimum(y1, img_ref[1])
    x0_ref[...] = x0
    y0_ref[...] = y0
    x1_ref[...] = x1
    y1_ref[...] = y1
    ar_ref[...] = jnp.maximum(x1 - x0, 0.0) * jnp.maximum(y1 - y0, 0.0)
    s_ref[...] = s_in_ref[...]

    lin = (jax.lax.broadcasted_iota(jnp.int32, (ROWS, LANES), 0) * LANES
           + jax.lax.broadcasted_iota(jnp.int32, (ROWS, LANES), 1))
    klin = (jax.lax.broadcasted_iota(jnp.int32, (8, LANES), 0) * LANES
            + jax.lax.broadcasted_iota(jnp.int32, (8, LANES), 1))

    def body(i, keep):
        s = s_ref[...]
        m = jnp.max(s)
        valid = m > NEG_INF
        idx = jnp.min(jnp.where(s == m, lin, jnp.int32(2**30)))
        sel = lin == idx
        bx0 = jnp.sum(jnp.where(sel, x0_ref[...], 0.0))
        by0 = jnp.sum(jnp.where(sel, y0_ref[...], 0.0))
        bx1 = jnp.sum(jnp.where(sel, x1_ref[...], 0.0))
        by1 = jnp.sum(jnp.where(sel, y1_ref[...], 0.0))
        barea = jnp.maximum(bx1 - bx0, 0.0) * jnp.maximum(by1 - by0, 0.0)
        yy1 = jnp.maximum(bx0, x0_ref[...])
        xx1 = jnp.maximum(by0, y0_ref[...])
        yy2 = jnp.minimum(bx1, x1_ref[...])
        xx2 = jnp.minimum(by1, y1_ref[...])
        inter = jnp.maximum(yy2 - yy1, 0.0) * jnp.maximum(xx2 - xx1, 0.0)
        union = (barea + ar_ref[...]) - inter
        iou = jnp.where(union > 0.0, inter / union, 0.0)
        supp = (iou > IOU_THR) & valid
        s_ref[...] = jnp.where(supp | sel, NEG_INF, s)
        kval = jnp.where(valid, idx, jnp.int32(-1)).astype(jnp.int32)
        return jnp.where(klin == i, kval, keep)

    keep0 = jnp.full((8, LANES), -1, dtype=jnp.int32)
    keep_ref[...] = jax.lax.fori_loop(0, NMS_OUT, body, keep0)


@functools.partial(jax.jit, static_argnames=())
def kernel(rpn_feature, anchors, img_sz, W_cls, b_cls, W_bbox, b_bbox):
    flat = rpn_feature.reshape(N_PIX, C_IN)
    prob, bbox = pl.pallas_call(
        _head_body,
        out_shape=(
            jax.ShapeDtypeStruct((N_PIX, 2 * ANCHORS_NUM), jnp.float32),
            jax.ShapeDtypeStruct((N_PIX, 4 * ANCHORS_NUM), jnp.float32),
        ),
    )(flat, W_cls, b_cls.reshape(1, -1), W_bbox, b_bbox.reshape(1, -1))

    fg = prob[:, ANCHORS_NUM:].reshape(-1)
    s_in = jnp.pad(fg, (0, N_PAD), constant_values=NEG_INF).reshape(ROWS, LANES)
    anc = jnp.pad(anchors.T, ((0, 0), (0, N_PAD))).reshape(4, ROWS, LANES)
    pred_t = bbox.reshape(N_BOX, 4).T
    tsel = jnp.concatenate([pred_t[0:2], pred_t[3:4]], axis=0)
    tpl = jnp.pad(tsel, ((0, 0), (0, N_PAD))).reshape(3, ROWS, LANES)

    keep = pl.pallas_call(
        _nms_body,
        out_shape=jax.ShapeDtypeStruct((8, LANES), jnp.int32),
        in_specs=[
            pl.BlockSpec(),
            pl.BlockSpec(),
            pl.BlockSpec(),
            pl.BlockSpec(memory_space=pltpu.SMEM),
        ],
        scratch_shapes=[pltpu.VMEM((ROWS, LANES), jnp.float32)] * 6,
    )(anc, tpl, s_in, img_sz)
    return keep.reshape(-1)[:NMS_OUT]
